# baseline (speedup 1.0000x reference)
import functools

import jax
import jax.numpy as jnp
from jax import lax
from jax.experimental import pallas as pl
from jax.experimental.pallas import tpu as pltpu
from jax.experimental.pallas import tpu_sc as plsc

_N = 10000
_D = 128
_E = 320000

_NTILES = 16
_EPT = 20480
_EPAD = _NTILES * _EPT
_K = 64
_NCH = _EPT // _K
_RG = 5
_RI = 10
_UNROLL = 10
_ROWS_PT = 624
_ROWS_LAST = _N - 15 * _ROWS_PT

_ROW_BLK = 2000


def _dense_body(xl_ref, xr_ref, wrl_lr_ref, wrt_lr_ref, wrl_rl_ref,
                wrt_rl_ref, b_lr_ref, b_rl_ref,
                yl_ref, yr_ref, rootl_ref, rootr_ref):
    dn = (((1,), (1,)), ((), ()))
    xl = xl_ref[...]
    xr = xr_ref[...]
    yl_ref[...] = lax.dot_general(xl, wrl_lr_ref[...], dn,
                                  preferred_element_type=jnp.float32)
    yr_ref[...] = lax.dot_general(xr, wrl_rl_ref[...], dn,
                                  preferred_element_type=jnp.float32)
    rootl_ref[...] = lax.dot_general(xl, wrt_rl_ref[...], dn,
                                     preferred_element_type=jnp.float32) + b_rl_ref[...]
    rootr_ref[...] = lax.dot_general(xr, wrt_lr_ref[...], dn,
                                     preferred_element_type=jnp.float32) + b_lr_ref[...]


def _dense(left, right, w_rel_lr, w_root_lr, w_rel_rl, w_root_rl, b_lr, b_rl):
    grid = (_N // _ROW_BLK,)
    row_spec = pl.BlockSpec((_ROW_BLK, _D), lambda i: (i, 0))
    full_spec = pl.BlockSpec((_D, _D), lambda i: (0, 0))
    bias_spec = pl.BlockSpec((1, _D), lambda i: (0, 0))
    out_sd = jax.ShapeDtypeStruct((_N, _D), jnp.float32)
    return pl.pallas_call(
        _dense_body,
        grid=grid,
        in_specs=[row_spec, row_spec, full_spec, full_spec, full_spec,
                  full_spec, bias_spec, bias_spec],
        out_specs=[row_spec, row_spec, row_spec, row_spec],
        out_shape=[out_sd, out_sd, out_sd, out_sd],
    )(left, right, w_rel_lr, w_root_lr, w_rel_rl, w_root_rl,
      b_lr.reshape(1, _D), b_rl.reshape(1, _D))


def _sc_body(yl, yr, rootl, rootr, comb_r2l, comb_l2r,
             outl, outr,
             g0, g1, g2, g3, g4,
             ib0, ib1, ib2, ib3, ib4, ib5, ib6, ib7, ib8, ib9,
             accum,
             gs0, gs1, gs2, gs3, gs4,
             ga0, ga1, ga2, ga3, ga4,
             ss0, ss1, ss2, ss3, ss4,
             is0, is1, is2, is3, is4, is5, is6, is7, is8, is9):
    c = lax.axis_index("c")
    s = lax.axis_index("s")
    rbase = s * _ROWS_PT
    grow = (g0, g1, g2, g3, g4)
    ibuf = (ib0, ib1, ib2, ib3, ib4, ib5, ib6, ib7, ib8, ib9)
    gsem = (gs0, gs1, gs2, gs3, gs4)
    gsem2 = (ga0, ga1, ga2, ga3, ga4)
    ssem = (ss0, ss1, ss2, ss3, ss4)
    isem = (is0, is1, is2, is3, is4, is5, is6, is7, is8, is9)

    def run_direction(y_hbm, root_hbm, comb_hbm, out_hbm):
        @pl.when(s < 15)
        def _():
            pltpu.sync_copy(root_hbm.at[pl.ds(rbase, _ROWS_PT)],
                            accum.at[pl.ds(rbase, _ROWS_PT)])

        @pl.when(s == 15)
        def _():
            pltpu.sync_copy(root_hbm.at[pl.ds(15 * _ROWS_PT, _ROWS_LAST)],
                            accum.at[pl.ds(15 * _ROWS_PT, _ROWS_LAST)])

        plsc.subcore_barrier()

        cbase = s * _NCH

        def idx_load(cc, q):
            pltpu.async_copy(comb_hbm.at[cbase + cc], ibuf[q], isem[q])

        def idx_wait(cc, q):
            pltpu.make_async_copy(comb_hbm.at[cbase + cc], ibuf[q],
                                  isem[q]).wait()

        def gather_start(b, q):
            pltpu.async_copy(y_hbm.at[ibuf[q].at[0, pl.ds(0, _K // 2)]],
                             grow[b].at[pl.ds(0, _K // 2)], gsem[b])
            pltpu.async_copy(y_hbm.at[ibuf[q].at[0, pl.ds(_K // 2, _K // 2)]],
                             grow[b].at[pl.ds(_K // 2, _K // 2)], gsem2[b])

        def gather_wait(b):
            pltpu.make_async_copy(y_hbm.at[ibuf[0].at[0, pl.ds(0, _K // 2)]],
                                  grow[b].at[pl.ds(0, _K // 2)],
                                  gsem[b]).wait()
            pltpu.make_async_copy(y_hbm.at[ibuf[0].at[0, pl.ds(0, _K // 2)]],
                                  grow[b].at[pl.ds(_K // 2, _K // 2)],
                                  gsem2[b]).wait()

        def scatter_start(b, q):
            pltpu.async_copy(grow[b], accum.at[ibuf[q].at[1]],
                             ssem[b], add=True)

        def scatter_wait(b):
            pltpu.make_async_copy(grow[b], accum.at[ibuf[0].at[1]],
                                  ssem[b]).wait()

        def scale_chunk(b, q):
            def edge(e, carry2):
                w16i = ibuf[q][2, pl.ds((e >> 4) << 4, 16)]
                w16 = lax.bitcast_convert_type(w16i, jnp.float32)
                wv = lax.gather(
                    w16,
                    lax.full((16, 1), e & 15, jnp.int32),
                    lax.GatherDimensionNumbers(offset_dims=(),
                                               collapsed_slice_dims=(0,),
                                               start_index_map=(0,)),
                    (1,),
                    mode=lax.GatherScatterMode.PROMISE_IN_BOUNDS)
                for d in range(_D // 16):
                    grow[b][e, pl.ds(d * 16, 16)] = (
                        grow[b][e, pl.ds(d * 16, 16)] * wv)
                return carry2

            lax.fori_loop(0, _K, edge, 0, unroll=2)

        for q in range(8):
            idx_load(q, q)
        for q in range(3):
            idx_wait(q, q)
            gather_start(q, q)

        def body(i, carry):
            for u in range(_UNROLL):
                cc = _UNROLL * i + u
                b = u % _RG
                b2 = (u - 2) % _RG
                b3 = (u + 3) % _RG
                q3 = (u + 3) % _RI
                q8 = (u + 8) % _RI

                gather_wait(b)

                @pl.when(cc >= 2)
                def _():
                    scatter_wait(b2)

                scale_chunk(b, u)
                scatter_start(b, u)

                @pl.when(cc + 3 < _NCH)
                def _():
                    idx_wait(cc + 3, q3)
                    gather_start(b3, q3)

                @pl.when(cc + 8 < _NCH)
                def _():
                    idx_load(cc + 8, q8)
            return carry

        lax.fori_loop(0, _NCH // _UNROLL, body, 0)
        scatter_wait((_NCH - 2) % _RG)
        scatter_wait((_NCH - 1) % _RG)
        plsc.subcore_barrier()

        @pl.when(s < 15)
        def _():
            pltpu.sync_copy(accum.at[pl.ds(rbase, _ROWS_PT)],
                            out_hbm.at[pl.ds(rbase, _ROWS_PT)])

        @pl.when(s == 15)
        def _():
            pltpu.sync_copy(accum.at[pl.ds(15 * _ROWS_PT, _ROWS_LAST)],
                            out_hbm.at[pl.ds(15 * _ROWS_PT, _ROWS_LAST)])

    @pl.when(c == 0)
    def _():
        run_direction(yr, rootl, comb_r2l, outl)

    @pl.when(c == 1)
    def _():
        run_direction(yl, rootr, comb_l2r, outr)


_sc_scatter = functools.partial(
    pl.kernel,
    out_type=(jax.ShapeDtypeStruct((_N, _D), jnp.float32),
              jax.ShapeDtypeStruct((_N, _D), jnp.float32)),
    mesh=plsc.VectorSubcoreMesh(core_axis_name="c", subcore_axis_name="s"),
    scratch_types=(
        (pltpu.VMEM((_K, _D), jnp.float32),) * 5
        + (pltpu.VMEM((3, _K), jnp.int32),) * 10
        + (pltpu.VMEM_SHARED((_N, _D), jnp.float32),)
        + (pltpu.SemaphoreType.DMA,) * 25
    ),
)(_sc_body)


def kernel(left_feas, right_feas, edge_index, edge_weight,
           W_rel_lr, b_lr, W_root_lr, W_rel_rl, b_rl, W_root_rl):
    yl, yr, rootl, rootr = _dense(left_feas, right_feas, W_rel_lr,
                                  W_root_lr, W_rel_rl, W_root_rl, b_lr, b_rl)
    pad = _EPAD - _E
    e0 = jnp.concatenate([edge_index[0], jnp.zeros((pad,), jnp.int32)])
    e1 = jnp.concatenate([edge_index[1], jnp.zeros((pad,), jnp.int32)])
    ew = jnp.concatenate([edge_weight, jnp.zeros((pad,), jnp.float32)])
    e0 = e0.reshape(_NTILES * _NCH, _K)
    e1 = e1.reshape(_NTILES * _NCH, _K)
    wi = lax.bitcast_convert_type(ew, jnp.int32).reshape(_NTILES * _NCH, _K)
    comb_r2l = jnp.stack([e0, e1, wi], axis=1)
    comb_l2r = jnp.stack([e1, e0, wi], axis=1)
    left_updated, right_updated = _sc_scatter(yl, yr, rootl, rootr,
                                              comb_r2l, comb_l2r)
    return (left_updated, right_updated)


# speedup vs baseline: 1.0001x; 1.0001x over previous
import functools

import jax
import jax.numpy as jnp
from jax import lax
from jax.experimental import pallas as pl
from jax.experimental.pallas import tpu as pltpu
from jax.experimental.pallas import tpu_sc as plsc

_N = 10000
_D = 128
_E = 320000

_NTILES = 16
_EPT = 20480
_EPAD = _NTILES * _EPT
_K = 64
_NCH = _EPT // _K
_RG = 5
_RI = 10
_UNROLL = 10
_ROWS_PT = 624
_ROWS_LAST = _N - 15 * _ROWS_PT

_ROW_BLK = 2000


def _dense_body(xl_ref, xr_ref, wrl_lr_ref, wrt_lr_ref, wrl_rl_ref,
                wrt_rl_ref, b_lr_ref, b_rl_ref,
                yl_ref, yr_ref, rootl_ref, rootr_ref):
    dn = (((1,), (1,)), ((), ()))
    xl = xl_ref[...]
    xr = xr_ref[...]
    yl_ref[...] = lax.dot_general(xl, wrl_lr_ref[...], dn,
                                  preferred_element_type=jnp.float32)
    yr_ref[...] = lax.dot_general(xr, wrl_rl_ref[...], dn,
                                  preferred_element_type=jnp.float32)
    rootl_ref[...] = lax.dot_general(xl, wrt_rl_ref[...], dn,
                                     preferred_element_type=jnp.float32) + b_rl_ref[...]
    rootr_ref[...] = lax.dot_general(xr, wrt_lr_ref[...], dn,
                                     preferred_element_type=jnp.float32) + b_lr_ref[...]


def _dense(left, right, w_rel_lr, w_root_lr, w_rel_rl, w_root_rl, b_lr, b_rl):
    grid = (_N // _ROW_BLK,)
    row_spec = pl.BlockSpec((_ROW_BLK, _D), lambda i: (i, 0))
    full_spec = pl.BlockSpec((_D, _D), lambda i: (0, 0))
    bias_spec = pl.BlockSpec((1, _D), lambda i: (0, 0))
    out_sd = jax.ShapeDtypeStruct((_N, _D), jnp.float32)
    return pl.pallas_call(
        _dense_body,
        grid=grid,
        in_specs=[row_spec, row_spec, full_spec, full_spec, full_spec,
                  full_spec, bias_spec, bias_spec],
        out_specs=[row_spec, row_spec, row_spec, row_spec],
        out_shape=[out_sd, out_sd, out_sd, out_sd],
    )(left, right, w_rel_lr, w_root_lr, w_rel_rl, w_root_rl,
      b_lr.reshape(1, _D), b_rl.reshape(1, _D))


def _sc_body(yl, yr, rootl, rootr, comb_r2l, comb_l2r,
             outl, outr,
             g0, g1, g2, g3, g4,
             ib0, ib1, ib2, ib3, ib4, ib5, ib6, ib7, ib8, ib9,
             accum,
             gs0, gs1, gs2, gs3, gs4,
             ss0, ss1, ss2, ss3, ss4,
             is0, is1, is2, is3, is4, is5, is6, is7, is8, is9):
    c = lax.axis_index("c")
    s = lax.axis_index("s")
    rbase = s * _ROWS_PT
    grow = (g0, g1, g2, g3, g4)
    ibuf = (ib0, ib1, ib2, ib3, ib4, ib5, ib6, ib7, ib8, ib9)
    gsem = (gs0, gs1, gs2, gs3, gs4)
    ssem = (ss0, ss1, ss2, ss3, ss4)
    isem = (is0, is1, is2, is3, is4, is5, is6, is7, is8, is9)

    def run_direction(y_hbm, root_hbm, comb_hbm, out_hbm):
        @pl.when(s < 15)
        def _():
            pltpu.sync_copy(root_hbm.at[pl.ds(rbase, _ROWS_PT)],
                            accum.at[pl.ds(rbase, _ROWS_PT)])

        @pl.when(s == 15)
        def _():
            pltpu.sync_copy(root_hbm.at[pl.ds(15 * _ROWS_PT, _ROWS_LAST)],
                            accum.at[pl.ds(15 * _ROWS_PT, _ROWS_LAST)])

        plsc.subcore_barrier()

        cbase = s * _NCH

        def idx_load(cc, q):
            pltpu.async_copy(comb_hbm.at[cbase + cc], ibuf[q], isem[q])

        def idx_wait(cc, q):
            pltpu.make_async_copy(comb_hbm.at[cbase + cc], ibuf[q],
                                  isem[q]).wait()

        def gather_start(b, q):
            pltpu.async_copy(y_hbm.at[ibuf[q].at[0]], grow[b], gsem[b])

        def gather_wait(b):
            pltpu.make_async_copy(y_hbm.at[ibuf[0].at[0]], grow[b],
                                  gsem[b]).wait()

        def scatter_start(b, q):
            pltpu.async_copy(grow[b], accum.at[ibuf[q].at[1]],
                             ssem[b], add=True)

        def scatter_wait(b):
            pltpu.make_async_copy(grow[b], accum.at[ibuf[0].at[1]],
                                  ssem[b]).wait()

        def scale_chunk(b, q):
            def edge(e, carry2):
                w16i = ibuf[q][2, pl.ds((e >> 4) << 4, 16)]
                w16 = lax.bitcast_convert_type(w16i, jnp.float32)
                wv = lax.gather(
                    w16,
                    lax.full((16, 1), e & 15, jnp.int32),
                    lax.GatherDimensionNumbers(offset_dims=(),
                                               collapsed_slice_dims=(0,),
                                               start_index_map=(0,)),
                    (1,),
                    mode=lax.GatherScatterMode.PROMISE_IN_BOUNDS)
                for d in range(_D // 16):
                    grow[b][e, pl.ds(d * 16, 16)] = (
                        grow[b][e, pl.ds(d * 16, 16)] * wv)
                return carry2

            lax.fori_loop(0, _K, edge, 0, unroll=2)

        for q in range(8):
            idx_load(q, q)
        for q in range(3):
            idx_wait(q, q)
            gather_start(q, q)

        def body(i, carry):
            for u in range(_UNROLL):
                cc = _UNROLL * i + u
                b = u % _RG
                b2 = (u - 2) % _RG
                b3 = (u + 3) % _RG
                q3 = (u + 3) % _RI
                q8 = (u + 8) % _RI

                gather_wait(b)

                @pl.when(cc >= 2)
                def _():
                    scatter_wait(b2)

                scale_chunk(b, u)
                scatter_start(b, u)

                @pl.when(cc + 3 < _NCH)
                def _():
                    idx_wait(cc + 3, q3)
                    gather_start(b3, q3)

                @pl.when(cc + 8 < _NCH)
                def _():
                    idx_load(cc + 8, q8)
            return carry

        lax.fori_loop(0, _NCH // _UNROLL, body, 0)
        scatter_wait((_NCH - 2) % _RG)
        scatter_wait((_NCH - 1) % _RG)
        plsc.subcore_barrier()

        @pl.when(s < 15)
        def _():
            pltpu.sync_copy(accum.at[pl.ds(rbase, _ROWS_PT)],
                            out_hbm.at[pl.ds(rbase, _ROWS_PT)])

        @pl.when(s == 15)
        def _():
            pltpu.sync_copy(accum.at[pl.ds(15 * _ROWS_PT, _ROWS_LAST)],
                            out_hbm.at[pl.ds(15 * _ROWS_PT, _ROWS_LAST)])

    @pl.when(c == 0)
    def _():
        run_direction(yr, rootl, comb_r2l, outl)

    @pl.when(c == 1)
    def _():
        run_direction(yl, rootr, comb_l2r, outr)


_sc_scatter = functools.partial(
    pl.kernel,
    out_type=(jax.ShapeDtypeStruct((_N, _D), jnp.float32),
              jax.ShapeDtypeStruct((_N, _D), jnp.float32)),
    mesh=plsc.VectorSubcoreMesh(core_axis_name="c", subcore_axis_name="s"),
    scratch_types=(
        (pltpu.VMEM((_K, _D), jnp.float32),) * 5
        + (pltpu.VMEM((3, _K), jnp.int32),) * 10
        + (pltpu.VMEM_SHARED((_N, _D), jnp.float32),)
        + (pltpu.SemaphoreType.DMA,) * 20
    ),
)(_sc_body)


def kernel(left_feas, right_feas, edge_index, edge_weight,
           W_rel_lr, b_lr, W_root_lr, W_rel_rl, b_rl, W_root_rl):
    yl, yr, rootl, rootr = _dense(left_feas, right_feas, W_rel_lr,
                                  W_root_lr, W_rel_rl, W_root_rl, b_lr, b_rl)
    pad = _EPAD - _E
    e0 = jnp.concatenate([edge_index[0], jnp.zeros((pad,), jnp.int32)])
    e1 = jnp.concatenate([edge_index[1], jnp.zeros((pad,), jnp.int32)])
    ew = jnp.concatenate([edge_weight, jnp.zeros((pad,), jnp.float32)])
    e0 = e0.reshape(_NTILES * _NCH, _K)
    e1 = e1.reshape(_NTILES * _NCH, _K)
    wi = lax.bitcast_convert_type(ew, jnp.int32).reshape(_NTILES * _NCH, _K)
    comb_r2l = jnp.stack([e0, e1, wi], axis=1)
    comb_l2r = jnp.stack([e1, e0, wi], axis=1)
    left_updated, right_updated = _sc_scatter(yl, yr, rootl, rootr,
                                              comb_r2l, comb_l2r)
    return (left_updated, right_updated)


# speedup vs baseline: 1.2219x; 1.2219x over previous
import functools

import jax
import jax.numpy as jnp
from jax import lax
from jax.experimental import pallas as pl
from jax.experimental.pallas import tpu as pltpu
from jax.experimental.pallas import tpu_sc as plsc

_N = 10000
_D = 128
_E = 320000

_NTILES = 16
_EPT = 20480
_EPAD = _NTILES * _EPT
_K = 64
_NCH = _EPT // _K
_RG = 5
_RI = 10
_UNROLL = 10
_ROWS_PT = 624
_ROWS_LAST = _N - 15 * _ROWS_PT

_ROW_BLK = 2000


def _dense_body(xl_ref, xr_ref, wrl_lr_ref, wrt_lr_ref, wrl_rl_ref,
                wrt_rl_ref, b_lr_ref, b_rl_ref,
                yl_ref, yr_ref, rootl_ref, rootr_ref):
    dn = (((1,), (1,)), ((), ()))
    xl = xl_ref[...].astype(jnp.bfloat16)
    xr = xr_ref[...].astype(jnp.bfloat16)
    yl_ref[...] = lax.dot_general(xl, wrl_lr_ref[...].astype(jnp.bfloat16), dn,
                                  preferred_element_type=jnp.float32)
    yr_ref[...] = lax.dot_general(xr, wrl_rl_ref[...].astype(jnp.bfloat16), dn,
                                  preferred_element_type=jnp.float32)
    rootl_ref[...] = lax.dot_general(xl, wrt_rl_ref[...].astype(jnp.bfloat16), dn,
                                     preferred_element_type=jnp.float32) + b_rl_ref[...]
    rootr_ref[...] = lax.dot_general(xr, wrt_lr_ref[...].astype(jnp.bfloat16), dn,
                                     preferred_element_type=jnp.float32) + b_lr_ref[...]


def _dense(left, right, w_rel_lr, w_root_lr, w_rel_rl, w_root_rl, b_lr, b_rl):
    grid = (_N // _ROW_BLK,)
    row_spec = pl.BlockSpec((_ROW_BLK, _D), lambda i: (i, 0))
    full_spec = pl.BlockSpec((_D, _D), lambda i: (0, 0))
    bias_spec = pl.BlockSpec((1, _D), lambda i: (0, 0))
    out_sd = jax.ShapeDtypeStruct((_N, _D), jnp.float32)
    return pl.pallas_call(
        _dense_body,
        grid=grid,
        in_specs=[row_spec, row_spec, full_spec, full_spec, full_spec,
                  full_spec, bias_spec, bias_spec],
        out_specs=[row_spec, row_spec, row_spec, row_spec],
        out_shape=[out_sd, out_sd, out_sd, out_sd],
    )(left, right, w_rel_lr, w_root_lr, w_rel_rl, w_root_rl,
      b_lr.reshape(1, _D), b_rl.reshape(1, _D))


def _sc_body(yl, yr, rootl, rootr, comb,
             outl, outr,
             g0, g1, g2, g3, g4,
             ib0, ib1, ib2, ib3, ib4, ib5, ib6, ib7, ib8, ib9,
             accum,
             gs0, gs1, gs2, gs3, gs4,
             ss0, ss1, ss2, ss3, ss4,
             is0, is1, is2, is3, is4, is5, is6, is7, is8, is9):
    c = lax.axis_index("c")
    s = lax.axis_index("s")
    rbase = s * _ROWS_PT
    grow = (g0, g1, g2, g3, g4)
    ibuf = (ib0, ib1, ib2, ib3, ib4, ib5, ib6, ib7, ib8, ib9)
    gsem = (gs0, gs1, gs2, gs3, gs4)
    ssem = (ss0, ss1, ss2, ss3, ss4)
    isem = (is0, is1, is2, is3, is4, is5, is6, is7, is8, is9)

    def run_direction(y_hbm, root_hbm, comb_hbm, out_hbm, si, di):
        @pl.when(s < 15)
        def _():
            pltpu.sync_copy(root_hbm.at[pl.ds(rbase, _ROWS_PT)],
                            accum.at[pl.ds(rbase, _ROWS_PT)])

        @pl.when(s == 15)
        def _():
            pltpu.sync_copy(root_hbm.at[pl.ds(15 * _ROWS_PT, _ROWS_LAST)],
                            accum.at[pl.ds(15 * _ROWS_PT, _ROWS_LAST)])

        plsc.subcore_barrier()

        cbase = s * _NCH

        def idx_load(cc, q):
            pltpu.async_copy(comb_hbm.at[cbase + cc], ibuf[q], isem[q])

        def idx_wait(cc, q):
            pltpu.make_async_copy(comb_hbm.at[cbase + cc], ibuf[q],
                                  isem[q]).wait()

        def gather_start(b, q):
            pltpu.async_copy(y_hbm.at[ibuf[q].at[si]], grow[b], gsem[b])

        def gather_wait(b):
            pltpu.make_async_copy(y_hbm.at[ibuf[0].at[si]], grow[b],
                                  gsem[b]).wait()

        def scatter_start(b, q):
            pltpu.async_copy(grow[b], accum.at[ibuf[q].at[di]],
                             ssem[b], add=True)

        def scatter_wait(b):
            pltpu.make_async_copy(grow[b], accum.at[ibuf[0].at[di]],
                                  ssem[b]).wait()

        def scale_chunk(b, q):
            def edge(e, carry2):
                w16i = ibuf[q][2, pl.ds((e >> 4) << 4, 16)]
                w16 = lax.bitcast_convert_type(w16i, jnp.float32)
                wv = lax.gather(
                    w16,
                    lax.full((16, 1), e & 15, jnp.int32),
                    lax.GatherDimensionNumbers(offset_dims=(),
                                               collapsed_slice_dims=(0,),
                                               start_index_map=(0,)),
                    (1,),
                    mode=lax.GatherScatterMode.PROMISE_IN_BOUNDS)
                for d in range(_D // 16):
                    grow[b][e, pl.ds(d * 16, 16)] = (
                        grow[b][e, pl.ds(d * 16, 16)] * wv)
                return carry2

            lax.fori_loop(0, _K, edge, 0, unroll=2)

        for q in range(8):
            idx_load(q, q)
        for q in range(3):
            idx_wait(q, q)
            gather_start(q, q)

        def body(i, carry):
            for u in range(_UNROLL):
                cc = _UNROLL * i + u
                b = u % _RG
                b2 = (u - 2) % _RG
                b3 = (u + 3) % _RG
                q3 = (u + 3) % _RI
                q8 = (u + 8) % _RI

                gather_wait(b)

                @pl.when(cc >= 2)
                def _():
                    scatter_wait(b2)

                scale_chunk(b, u)
                scatter_start(b, u)

                @pl.when(cc + 3 < _NCH)
                def _():
                    idx_wait(cc + 3, q3)
                    gather_start(b3, q3)

                @pl.when(cc + 8 < _NCH)
                def _():
                    idx_load(cc + 8, q8)
            return carry

        lax.fori_loop(0, _NCH // _UNROLL, body, 0)
        scatter_wait((_NCH - 2) % _RG)
        scatter_wait((_NCH - 1) % _RG)
        plsc.subcore_barrier()

        @pl.when(s < 15)
        def _():
            pltpu.sync_copy(accum.at[pl.ds(rbase, _ROWS_PT)],
                            out_hbm.at[pl.ds(rbase, _ROWS_PT)])

        @pl.when(s == 15)
        def _():
            pltpu.sync_copy(accum.at[pl.ds(15 * _ROWS_PT, _ROWS_LAST)],
                            out_hbm.at[pl.ds(15 * _ROWS_PT, _ROWS_LAST)])

    @pl.when(c == 0)
    def _():
        run_direction(yr, rootl, comb, outl, 0, 1)

    @pl.when(c == 1)
    def _():
        run_direction(yl, rootr, comb, outr, 1, 0)


_sc_scatter = functools.partial(
    pl.kernel,
    out_type=(jax.ShapeDtypeStruct((_N, _D), jnp.float32),
              jax.ShapeDtypeStruct((_N, _D), jnp.float32)),
    mesh=plsc.VectorSubcoreMesh(core_axis_name="c", subcore_axis_name="s"),
    scratch_types=(
        (pltpu.VMEM((_K, _D), jnp.float32),) * 5
        + (pltpu.VMEM((3, _K), jnp.int32),) * 10
        + (pltpu.VMEM_SHARED((_N, _D), jnp.float32),)
        + (pltpu.SemaphoreType.DMA,) * 20
    ),
)(_sc_body)


def kernel(left_feas, right_feas, edge_index, edge_weight,
           W_rel_lr, b_lr, W_root_lr, W_rel_rl, b_rl, W_root_rl):
    yl, yr, rootl, rootr = _dense(left_feas, right_feas, W_rel_lr,
                                  W_root_lr, W_rel_rl, W_root_rl, b_lr, b_rl)
    pad = _EPAD - _E
    e0 = jnp.concatenate([edge_index[0], jnp.zeros((pad,), jnp.int32)])
    e1 = jnp.concatenate([edge_index[1], jnp.zeros((pad,), jnp.int32)])
    ew = jnp.concatenate([edge_weight, jnp.zeros((pad,), jnp.float32)])
    e0 = e0.reshape(_NTILES * _NCH, _K)
    e1 = e1.reshape(_NTILES * _NCH, _K)
    wi = lax.bitcast_convert_type(ew, jnp.int32).reshape(_NTILES * _NCH, _K)
    comb = jnp.stack([e0, e1, wi], axis=1)
    left_updated, right_updated = _sc_scatter(yl, yr, rootl, rootr, comb)
    return (left_updated, right_updated)
